# Initial kernel scaffold; baseline (speedup 1.0000x reference)
import jax
import jax.numpy as jnp
from jax.experimental import pallas as pl


def kernel(x, duty_cycles):
    raise NotImplementedError("write your pallas kernel here")


# speedup vs baseline: 19.6549x; 19.6549x over previous
import functools

import jax
import jax.numpy as jnp
from jax.experimental import pallas as pl

_PERCENT_ON = 0.1
_BOOST_STRENGTH = 1.0


def _body(x_ref, duty_ref, o_ref, *, k):
    int_min = jnp.int32(-(2 ** 31))
    x = x_ref[...]
    duty = duty_ref[...]
    n = x.shape[-1]
    bf = jnp.exp((jnp.float32(k / n) - duty) * jnp.float32(_BOOST_STRENGTH))
    boosted = x * bf
    bits = jax.lax.bitcast_convert_type(boosted, jnp.int32)
    skey = jnp.where(bits >= 0, bits, bits ^ jnp.int32(0x7FFFFFFF))

    def step(i, v):
        b = 31 - i
        cand = v | (jnp.int32(1) << b)
        thr = cand ^ int_min
        cnt = jnp.sum((skey >= thr).astype(jnp.int32), axis=1, keepdims=True)
        return jnp.where(cnt >= k, cand, v)

    v0 = jnp.zeros((x.shape[0], 1), jnp.int32)
    v = jax.lax.fori_loop(0, 32, step, v0)
    thr = v ^ int_min
    o_ref[...] = jnp.where(skey >= thr, x, jnp.float32(0.0))


@jax.jit
def kernel(x, duty_cycles):
    b, n = x.shape
    k = int(round(n * _PERCENT_ON))
    rows = 8
    duty2 = duty_cycles.reshape(1, n)
    return pl.pallas_call(
        functools.partial(_body, k=k),
        grid=(b // rows,),
        in_specs=[
            pl.BlockSpec((rows, n), lambda i: (i, 0)),
            pl.BlockSpec((1, n), lambda i: (0, 0)),
        ],
        out_specs=pl.BlockSpec((rows, n), lambda i: (i, 0)),
        out_shape=jax.ShapeDtypeStruct((b, n), jnp.float32),
    )(x, duty2)
